# baseline (speedup 1.0000x reference)
import functools

import jax
import jax.numpy as jnp
from jax import lax
from jax.experimental import pallas as pl
from jax.experimental.pallas import tpu as pltpu
from jax.experimental.pallas import tpu_sc as plsc

EMBED_DIM = 32
BATCH_SIZE = 16384

NUM_CORES = 2
NUM_SUBCORES = 16
LANES = 16
NUM_WORKERS = NUM_CORES * NUM_SUBCORES
B_PER_W = BATCH_SIZE // NUM_WORKERS
NUM_GROUPS = B_PER_W // LANES

_mesh = plsc.VectorSubcoreMesh(core_axis_name="c", subcore_axis_name="s")


@functools.partial(
    pl.kernel,
    mesh=_mesh,
    out_type=jax.ShapeDtypeStruct((BATCH_SIZE,), jnp.float32),
    compiler_params=pltpu.CompilerParams(
        needs_layout_passes=False, use_tc_tiling_on_sc=False),
    scratch_types=[
        pltpu.VMEM((B_PER_W,), jnp.int32),
        pltpu.VMEM((B_PER_W,), jnp.int32),
        pltpu.VMEM((B_PER_W, EMBED_DIM), jnp.float32),
        pltpu.VMEM((B_PER_W, EMBED_DIM), jnp.float32),
        pltpu.VMEM((B_PER_W,), jnp.float32),
        pltpu.VMEM((B_PER_W,), jnp.float32),
        pltpu.VMEM((B_PER_W,), jnp.float32),
        pltpu.VMEM((LANES * LANES,), jnp.float32),
        pltpu.SemaphoreType.DMA,
        pltpu.SemaphoreType.DMA,
        pltpu.SemaphoreType.DMA,
        pltpu.SemaphoreType.DMA,
    ],
)
def _mf_kernel(user_hbm, movie_hbm, ue_hbm, me_hbm, ub_hbm, mb_hbm, out_hbm,
               uidx_v, midx_v, urows_v, mrows_v, ubias_v, mbias_v, acc_v,
               part_v, sem_u, sem_m, sem_ub, sem_mb):
    wid = lax.axis_index("s") * NUM_CORES + lax.axis_index("c")
    base = wid * B_PER_W

    pltpu.sync_copy(user_hbm.at[pl.ds(base, B_PER_W)], uidx_v)
    pltpu.sync_copy(movie_hbm.at[pl.ds(base, B_PER_W)], midx_v)

    dma_u = pltpu.async_copy(ue_hbm.at[uidx_v], urows_v, sem_u)
    dma_m = pltpu.async_copy(me_hbm.at[midx_v], mrows_v, sem_m)
    dma_ub = pltpu.async_copy(ub_hbm.at[uidx_v], ubias_v, sem_ub)
    dma_mb = pltpu.async_copy(mb_hbm.at[midx_v], mbias_v, sem_mb)
    dma_ub.wait()
    dma_mb.wait()
    dma_u.wait()
    dma_m.wait()

    lanes = lax.iota(jnp.int32, LANES)
    HALF = EMBED_DIM // 2
    row_base = lanes * LANES

    def group_body(g, carry):
        r = g * LANES
        for i in range(LANES):
            u0 = urows_v[r + i, pl.ds(0, HALF)]
            u1 = urows_v[r + i, pl.ds(HALF, HALF)]
            m0 = mrows_v[r + i, pl.ds(0, HALF)]
            m1 = mrows_v[r + i, pl.ds(HALF, HALF)]
            part_v[pl.ds(i * LANES, LANES)] = u0 * m0 + u1 * m1
        acc = ubias_v[pl.ds(r, LANES)] + mbias_v[pl.ds(r, LANES)]
        for k in range(LANES):
            acc = acc + plsc.load_gather(part_v, [row_base + k])
        acc_v[pl.ds(r, LANES)] = acc
        return carry

    lax.fori_loop(0, NUM_GROUPS, group_body, 0)

    pltpu.sync_copy(acc_v, out_hbm.at[pl.ds(base, B_PER_W)])


def kernel(user, movie, user_embedding, movie_embedding, user_bias, movie_bias):
    return _mf_kernel(
        user.astype(jnp.int32),
        movie.astype(jnp.int32),
        user_embedding,
        movie_embedding,
        user_bias.reshape(-1),
        movie_bias.reshape(-1),
    )
